# baseline (speedup 1.0000x reference)
import functools

import jax
import jax.numpy as jnp
from jax import lax
from jax.experimental import pallas as pl
from jax.experimental.pallas import tpu as pltpu
from jax.experimental.pallas import tpu_sc as plsc

NTOK = 2048
HID = 1024
INTER = 2048
NEXP = 8
TOPK = 2
NPAIR = NTOK * TOPK
BLK = 256
NBLK = NPAIR // BLK + (NEXP - 1)
PTOT = NBLK * BLK
IBLK = 512
NICH = INTER // IBLK

NWORK = 32
CHD = 64
CHC = 32


def _router_body(x_ref, gw_ref, logits_ref, w_ref, pos_ref, meta_ref):
    x = x_ref[...]
    gw = gw_ref[...]
    logits_ref[...] = lax.dot_general(
        x, gw, (((1,), (1,)), ((), ())), preferred_element_type=jnp.float32)
    lt = lax.dot_general(
        gw, x, (((1,), (1,)), ((), ())), preferred_element_type=jnp.float32)
    m = jnp.max(lt, axis=0, keepdims=True)
    ex = jnp.exp(lt - m)
    p = ex / jnp.sum(ex, axis=0, keepdims=True)
    iota = lax.broadcasted_iota(jnp.int32, (NEXP, NTOK), 0).astype(jnp.float32)
    v1 = jnp.max(p, axis=0, keepdims=True)
    e1 = jnp.min(jnp.where(p == v1, iota, float(NEXP)), axis=0, keepdims=True)
    oh1 = (iota == e1).astype(jnp.float32)
    p2 = jnp.where(oh1 > 0, -1.0, p)
    v2 = jnp.max(p2, axis=0, keepdims=True)
    e2 = jnp.min(jnp.where(p2 == v2, iota, float(NEXP)), axis=0, keepdims=True)
    oh2 = (iota == e2).astype(jnp.float32)
    sw = v1 + v2
    w_ref[0:1, :] = v1 / sw
    w_ref[1:2, :] = v2 / sw

    ch = 128
    su = (lax.broadcasted_iota(jnp.int32, (ch, ch), 0)
          < lax.broadcasted_iota(jnp.int32, (ch, ch), 1)).astype(jnp.float32)

    def excl_cumsum(oh):
        parts = []
        carry = jnp.zeros((NEXP, 1), jnp.float32)
        for j in range(NTOK // ch):
            c = oh[:, j * ch:(j + 1) * ch]
            pref = lax.dot_general(c, su, (((1,), (0,)), ((), ())),
                                   preferred_element_type=jnp.float32)
            parts.append(pref + carry)
            carry = carry + jnp.sum(c, axis=1, keepdims=True)
        return jnp.concatenate(parts, axis=1), carry

    r1, cnt1 = excl_cumsum(oh1)
    r2, cnt2 = excl_cumsum(oh2)
    r2 = r2 + cnt1
    counts = cnt1 + cnt2
    pb = jnp.floor((counts + float(BLK - 1)) * (1.0 / BLK))
    ls = (lax.broadcasted_iota(jnp.int32, (NEXP, NEXP), 1)
          < lax.broadcasted_iota(jnp.int32, (NEXP, NEXP), 0)).astype(jnp.float32)
    sb = lax.dot_general(ls, pb, (((1,), (0,)), ((), ())),
                         preferred_element_type=jnp.float32)
    start_slot = sb * float(BLK)
    pos_a = jnp.sum(oh1 * (start_slot + r1), axis=0, keepdims=True)
    pos_b = jnp.sum(oh2 * (start_slot + r2), axis=0, keepdims=True)
    pos_ref[0:1, :] = pos_a.astype(jnp.int32)
    pos_ref[1:2, :] = pos_b.astype(jnp.int32)

    used = jnp.sum(pb, axis=0, keepdims=True)
    bio = lax.broadcasted_iota(jnp.int32, (1, NBLK), 1).astype(jnp.float32)
    bcl = jnp.minimum(bio, used - 1.0)
    eid = jnp.sum((sb <= bcl).astype(jnp.float32), axis=0, keepdims=True) - 1.0
    meta_ref[0:1, 0:NBLK] = eid.astype(jnp.int32)
    meta_ref[0:1, NBLK:NBLK + 1] = used.astype(jnp.int32)


def _router(x, gate_w):
    return pl.pallas_call(
        _router_body,
        out_shape=[
            jax.ShapeDtypeStruct((NTOK, NEXP), jnp.float32),
            jax.ShapeDtypeStruct((TOPK, NTOK), jnp.float32),
            jax.ShapeDtypeStruct((TOPK, NTOK), jnp.int32),
            jax.ShapeDtypeStruct((1, NBLK + 1), jnp.int32),
        ],
    )(x, gate_w)


def _ffn_body(meta_ref, xs_ref, ws_ref, wg_ref, wu_ref, wd_ref, ys_ref, acc_ref):
    i = pl.program_id(0)
    b = pl.program_id(1)
    used = meta_ref[NBLK]

    @pl.when(b < used)
    def _():
        xb = xs_ref[...]
        g = lax.dot_general(xb, wg_ref[0], (((1,), (1,)), ((), ())),
                            preferred_element_type=jnp.float32)
        u = lax.dot_general(xb, wu_ref[0], (((1,), (1,)), ((), ())),
                            preferred_element_type=jnp.float32)
        h = g * lax.logistic(g) * u
        part = lax.dot_general(h, wd_ref[0], (((1,), (1,)), ((), ())),
                               preferred_element_type=jnp.float32)
        base = b * BLK

        @pl.when(i == 0)
        def _():
            acc_ref[pl.ds(base, BLK), :] = part

        @pl.when(i > 0)
        def _():
            acc_ref[pl.ds(base, BLK), :] += part

        @pl.when(i == NICH - 1)
        def _():
            ys_ref[...] = acc_ref[pl.ds(base, BLK), :] * ws_ref[:, 0:1]


def _ffn(meta, xs, ws2d, wg, wu, wd):
    def bmap(i, b, m):
        return (jnp.minimum(b, m[NBLK] - 1), 0)

    def wmap_g(i, b, m):
        return (m[b], i, 0)

    def wmap_d(i, b, m):
        return (m[b], 0, i)

    def omap(i, b, m):
        return (jnp.where(i == NICH - 1, jnp.minimum(b, m[NBLK] - 1), 0), 0)

    grid_spec = pltpu.PrefetchScalarGridSpec(
        num_scalar_prefetch=1,
        grid=(NICH, NBLK),
        in_specs=[
            pl.BlockSpec((BLK, HID), bmap),
            pl.BlockSpec((BLK, 128), bmap),
            pl.BlockSpec((1, IBLK, HID), wmap_g),
            pl.BlockSpec((1, IBLK, HID), wmap_g),
            pl.BlockSpec((1, HID, IBLK), wmap_d),
        ],
        out_specs=pl.BlockSpec((BLK, HID), omap),
        scratch_shapes=[pltpu.VMEM((PTOT, HID), jnp.float32)],
    )
    return pl.pallas_call(
        _ffn_body,
        grid_spec=grid_spec,
        out_shape=jax.ShapeDtypeStruct((PTOT, HID), jnp.float32),
    )(meta, xs, ws2d, wg, wu, wd)


def _dispatch_body(x_hbm, w16_hbm, pos_hbm, xs_hbm, ws_hbm,
                   idx_v, rows_v, wrow_v, sem1, sem2):
    wid = lax.axis_index("s") * 2 + lax.axis_index("c")
    base = wid * (NPAIR // NWORK)
    for j in range(NPAIR // NWORK // CHD):
        pbase = base + j * CHD
        pltpu.sync_copy(pos_hbm.at[pl.ds(pbase, CHD)], idx_v)
        tb = lax.rem(pbase, NTOK)
        pltpu.sync_copy(x_hbm.at[pl.ds(tb, CHD)], rows_v)
        pltpu.sync_copy(w16_hbm.at[pl.ds(pbase, CHD)], wrow_v)
        c1 = pltpu.async_copy(rows_v, xs_hbm.at[idx_v], sem1)
        c2 = pltpu.async_copy(wrow_v, ws_hbm.at[idx_v], sem2)
        c1.wait()
        c2.wait()


def _dispatch(x, w16, pos):
    mesh = plsc.VectorSubcoreMesh(core_axis_name="c", subcore_axis_name="s")
    call = pl.kernel(
        _dispatch_body,
        mesh=mesh,
        out_type=[
            jax.ShapeDtypeStruct((PTOT, HID), jnp.float32),
            jax.ShapeDtypeStruct((PTOT, 128), jnp.float32),
        ],
        scratch_types=[
            pltpu.VMEM((CHD,), jnp.int32),
            pltpu.VMEM((CHD, HID), jnp.float32),
            pltpu.VMEM((CHD, 128), jnp.float32),
            pltpu.SemaphoreType.DMA,
            pltpu.SemaphoreType.DMA,
        ],
    )
    return call(x, w16, pos)


def _combine_body(ys_hbm, pos_hbm, out_hbm, idx0_v, idx1_v, t0_v, t1_v, sem):
    wid = lax.axis_index("s") * 2 + lax.axis_index("c")
    base = wid * (NTOK // NWORK)
    for j in range(NTOK // NWORK // CHC):
        tb = base + j * CHC
        pltpu.sync_copy(pos_hbm.at[pl.ds(tb, CHC)], idx0_v)
        pltpu.sync_copy(pos_hbm.at[pl.ds(NTOK + tb, CHC)], idx1_v)
        pltpu.async_copy(ys_hbm.at[idx0_v], t0_v, sem).wait()
        pltpu.async_copy(ys_hbm.at[idx1_v], t1_v, sem).wait()

        @pl.loop(0, CHC)
        def _(r):
            @pl.loop(0, HID, step=16)
            def _(cc):
                slc = (pl.ds(r, 1), pl.ds(cc, 16))
                t0_v.at[*slc][...] = t0_v.at[*slc][...] + t1_v.at[*slc][...]

        pltpu.sync_copy(t0_v, out_hbm.at[pl.ds(tb, CHC)])


def _combine(ys, pos):
    mesh = plsc.VectorSubcoreMesh(core_axis_name="c", subcore_axis_name="s")
    call = pl.kernel(
        _combine_body,
        mesh=mesh,
        out_type=jax.ShapeDtypeStruct((NTOK, HID), jnp.float32),
        scratch_types=[
            pltpu.VMEM((CHC,), jnp.int32),
            pltpu.VMEM((CHC,), jnp.int32),
            pltpu.VMEM((CHC, HID), jnp.float32),
            pltpu.VMEM((CHC, HID), jnp.float32),
            pltpu.SemaphoreType.DMA,
        ],
    )
    return call(ys, pos)


def kernel(hidden_states, gate_w, wg, wu, wd):
    b, s, h = hidden_states.shape
    x = hidden_states.reshape(s, h)
    logits, wrow, posrow, meta = _router(x, gate_w)
    pos = posrow.reshape(NPAIR)
    w16 = jnp.broadcast_to(wrow.reshape(NPAIR, 1), (NPAIR, 128))
    xs, ws2d = _dispatch(x, w16, pos)
    ys = _ffn(meta.reshape(NBLK + 1), xs, ws2d, wg, wu, wd)
    final = _combine(ys, pos)
    return final.reshape(b, s, h), logits


# speedup vs baseline: 1.0073x; 1.0073x over previous
import functools

import jax
import jax.numpy as jnp
from jax import lax
from jax.experimental import pallas as pl
from jax.experimental.pallas import tpu as pltpu
from jax.experimental.pallas import tpu_sc as plsc

NTOK = 2048
HID = 1024
INTER = 2048
NEXP = 8
TOPK = 2
NPAIR = NTOK * TOPK
BLK = 256
NBLK = NPAIR // BLK + (NEXP - 1)
PTOT = NBLK * BLK
IBLK = 512
NICH = INTER // IBLK

NWORK = 32
CHD = 64
CHC = 32


def _router_body(x_ref, gw_ref, logits_ref, w_ref, pos_ref, meta_ref):
    x = x_ref[...]
    gw = gw_ref[...]
    logits_ref[...] = lax.dot_general(
        x, gw, (((1,), (1,)), ((), ())), preferred_element_type=jnp.float32)
    lt = lax.dot_general(
        gw, x, (((1,), (1,)), ((), ())), preferred_element_type=jnp.float32)
    m = jnp.max(lt, axis=0, keepdims=True)
    ex = jnp.exp(lt - m)
    p = ex / jnp.sum(ex, axis=0, keepdims=True)
    iota = lax.broadcasted_iota(jnp.int32, (NEXP, NTOK), 0).astype(jnp.float32)
    v1 = jnp.max(p, axis=0, keepdims=True)
    e1 = jnp.min(jnp.where(p == v1, iota, float(NEXP)), axis=0, keepdims=True)
    oh1 = (iota == e1).astype(jnp.float32)
    p2 = jnp.where(oh1 > 0, -1.0, p)
    v2 = jnp.max(p2, axis=0, keepdims=True)
    e2 = jnp.min(jnp.where(p2 == v2, iota, float(NEXP)), axis=0, keepdims=True)
    oh2 = (iota == e2).astype(jnp.float32)
    sw = v1 + v2
    w_ref[0:1, :] = v1 / sw
    w_ref[1:2, :] = v2 / sw

    ch = 128
    su = (lax.broadcasted_iota(jnp.int32, (ch, ch), 0)
          < lax.broadcasted_iota(jnp.int32, (ch, ch), 1)).astype(jnp.float32)

    def excl_cumsum(oh):
        parts = []
        carry = jnp.zeros((NEXP, 1), jnp.float32)
        for j in range(NTOK // ch):
            c = oh[:, j * ch:(j + 1) * ch]
            pref = lax.dot_general(c, su, (((1,), (0,)), ((), ())),
                                   preferred_element_type=jnp.float32)
            parts.append(pref + carry)
            carry = carry + jnp.sum(c, axis=1, keepdims=True)
        return jnp.concatenate(parts, axis=1), carry

    r1, cnt1 = excl_cumsum(oh1)
    r2, cnt2 = excl_cumsum(oh2)
    r2 = r2 + cnt1
    counts = cnt1 + cnt2
    pb = jnp.floor((counts + float(BLK - 1)) * (1.0 / BLK))
    ls = (lax.broadcasted_iota(jnp.int32, (NEXP, NEXP), 1)
          < lax.broadcasted_iota(jnp.int32, (NEXP, NEXP), 0)).astype(jnp.float32)
    sb = lax.dot_general(ls, pb, (((1,), (0,)), ((), ())),
                         preferred_element_type=jnp.float32)
    start_slot = sb * float(BLK)
    pos_a = jnp.sum(oh1 * (start_slot + r1), axis=0, keepdims=True)
    pos_b = jnp.sum(oh2 * (start_slot + r2), axis=0, keepdims=True)
    pos_ref[0:1, :] = pos_a.astype(jnp.int32)
    pos_ref[1:2, :] = pos_b.astype(jnp.int32)

    used = jnp.sum(pb, axis=0, keepdims=True)
    bio = lax.broadcasted_iota(jnp.int32, (1, NBLK), 1).astype(jnp.float32)
    bcl = jnp.minimum(bio, used - 1.0)
    eid = jnp.sum((sb <= bcl).astype(jnp.float32), axis=0, keepdims=True) - 1.0
    meta_ref[0:1, 0:NBLK] = eid.astype(jnp.int32)
    meta_ref[0:1, NBLK:NBLK + 1] = used.astype(jnp.int32)


def _router(x, gate_w):
    return pl.pallas_call(
        _router_body,
        out_shape=[
            jax.ShapeDtypeStruct((NTOK, NEXP), jnp.float32),
            jax.ShapeDtypeStruct((TOPK, NTOK), jnp.float32),
            jax.ShapeDtypeStruct((TOPK, NTOK), jnp.int32),
            jax.ShapeDtypeStruct((1, NBLK + 1), jnp.int32),
        ],
    )(x, gate_w)


def _ffn_body(meta_ref, xs_ref, ws_ref, wg_ref, wu_ref, wd_ref, ys_ref):
    b = pl.program_id(0)
    used = meta_ref[NBLK]

    @pl.when(b < used)
    def _():
        xb = xs_ref[...].astype(jnp.bfloat16)
        g = lax.dot_general(xb, wg_ref[0], (((1,), (1,)), ((), ())),
                            preferred_element_type=jnp.float32)
        u = lax.dot_general(xb, wu_ref[0], (((1,), (1,)), ((), ())),
                            preferred_element_type=jnp.float32)
        h = (g * lax.logistic(g) * u).astype(jnp.bfloat16)
        part = lax.dot_general(h, wd_ref[0], (((1,), (1,)), ((), ())),
                               preferred_element_type=jnp.float32)
        ys_ref[...] = part * ws_ref[:, 0:1]


def _ffn(meta, xs, ws2d, wg, wu, wd):
    def bmap(b, m):
        return (jnp.minimum(b, m[NBLK] - 1), 0)

    def wmap_g(b, m):
        return (m[b], 0, 0)

    def wmap_d(b, m):
        return (m[b], 0, 0)

    grid_spec = pltpu.PrefetchScalarGridSpec(
        num_scalar_prefetch=1,
        grid=(NBLK,),
        in_specs=[
            pl.BlockSpec((BLK, HID), bmap),
            pl.BlockSpec((BLK, 128), bmap),
            pl.BlockSpec((1, INTER, HID), wmap_g),
            pl.BlockSpec((1, INTER, HID), wmap_g),
            pl.BlockSpec((1, HID, INTER), wmap_d),
        ],
        out_specs=pl.BlockSpec((BLK, HID), bmap),
    )
    return pl.pallas_call(
        _ffn_body,
        grid_spec=grid_spec,
        out_shape=jax.ShapeDtypeStruct((PTOT, HID), jnp.float32),
    )(meta, xs, ws2d, wg, wu, wd)


def _dispatch_body(x_hbm, w16_hbm, pos_hbm, xs_hbm, ws_hbm,
                   idx_v, rows_v, wrow_v, sem1, sem2):
    wid = lax.axis_index("s") * 2 + lax.axis_index("c")
    base = wid * (NPAIR // NWORK)
    for j in range(NPAIR // NWORK // CHD):
        pbase = base + j * CHD
        pltpu.sync_copy(pos_hbm.at[pl.ds(pbase, CHD)], idx_v)
        tb = lax.rem(pbase, NTOK)
        pltpu.sync_copy(x_hbm.at[pl.ds(tb, CHD)], rows_v)
        pltpu.sync_copy(w16_hbm.at[pl.ds(pbase, CHD)], wrow_v)
        c1 = pltpu.async_copy(rows_v, xs_hbm.at[idx_v], sem1)
        c2 = pltpu.async_copy(wrow_v, ws_hbm.at[idx_v], sem2)
        c1.wait()
        c2.wait()


def _dispatch(x, w16, pos):
    mesh = plsc.VectorSubcoreMesh(core_axis_name="c", subcore_axis_name="s")
    call = pl.kernel(
        _dispatch_body,
        mesh=mesh,
        out_type=[
            jax.ShapeDtypeStruct((PTOT, HID), jnp.float32),
            jax.ShapeDtypeStruct((PTOT, 128), jnp.float32),
        ],
        scratch_types=[
            pltpu.VMEM((CHD,), jnp.int32),
            pltpu.VMEM((CHD, HID), jnp.float32),
            pltpu.VMEM((CHD, 128), jnp.float32),
            pltpu.SemaphoreType.DMA,
            pltpu.SemaphoreType.DMA,
        ],
    )
    return call(x, w16, pos)


def _combine_body(ys_hbm, pos_hbm, out_hbm, idx0_v, idx1_v, t0_v, t1_v, sem):
    wid = lax.axis_index("s") * 2 + lax.axis_index("c")
    base = wid * (NTOK // NWORK)
    for j in range(NTOK // NWORK // CHC):
        tb = base + j * CHC
        pltpu.sync_copy(pos_hbm.at[pl.ds(tb, CHC)], idx0_v)
        pltpu.sync_copy(pos_hbm.at[pl.ds(NTOK + tb, CHC)], idx1_v)
        pltpu.async_copy(ys_hbm.at[idx0_v], t0_v, sem).wait()
        pltpu.async_copy(ys_hbm.at[idx1_v], t1_v, sem).wait()

        @pl.loop(0, CHC)
        def _(r):
            @pl.loop(0, HID, step=16)
            def _(cc):
                slc = (pl.ds(r, 1), pl.ds(cc, 16))
                t0_v.at[*slc][...] = t0_v.at[*slc][...] + t1_v.at[*slc][...]

        pltpu.sync_copy(t0_v, out_hbm.at[pl.ds(tb, CHC)])


def _combine(ys, pos):
    mesh = plsc.VectorSubcoreMesh(core_axis_name="c", subcore_axis_name="s")
    call = pl.kernel(
        _combine_body,
        mesh=mesh,
        out_type=jax.ShapeDtypeStruct((NTOK, HID), jnp.float32),
        scratch_types=[
            pltpu.VMEM((CHC,), jnp.int32),
            pltpu.VMEM((CHC,), jnp.int32),
            pltpu.VMEM((CHC, HID), jnp.float32),
            pltpu.VMEM((CHC, HID), jnp.float32),
            pltpu.SemaphoreType.DMA,
        ],
    )
    return call(ys, pos)


def kernel(hidden_states, gate_w, wg, wu, wd):
    b, s, h = hidden_states.shape
    x = hidden_states.reshape(s, h)
    logits, wrow, posrow, meta = _router(x, gate_w)
    pos = posrow.reshape(NPAIR)
    w16 = jnp.broadcast_to(wrow.reshape(NPAIR, 1), (NPAIR, 128))
    xs, ws2d = _dispatch(x, w16, pos)
    ys = _ffn(meta.reshape(NBLK + 1), xs, ws2d,
              wg.astype(jnp.bfloat16), wu.astype(jnp.bfloat16),
              wd.astype(jnp.bfloat16))
    final = _combine(ys, pos)
    return final.reshape(b, s, h), logits


# speedup vs baseline: 1.2921x; 1.2827x over previous
import functools

import jax
import jax.numpy as jnp
from jax import lax
from jax.experimental import pallas as pl
from jax.experimental.pallas import tpu as pltpu
from jax.experimental.pallas import tpu_sc as plsc

NTOK = 2048
HID = 1024
INTER = 2048
NEXP = 8
TOPK = 2
NPAIR = NTOK * TOPK
BLK = 256
NBLK = NPAIR // BLK + (NEXP - 1)
PTOT = NBLK * BLK
IBLK = 512
NICH = INTER // IBLK

NWORK = 32
CHD = 64
CHC = 32


def _router_body(x_ref, gw_ref, logits_ref, w_ref, pos_ref, meta_ref):
    x = x_ref[...]
    gw = gw_ref[...]
    logits_ref[...] = lax.dot_general(
        x, gw, (((1,), (1,)), ((), ())), preferred_element_type=jnp.float32)
    lt = lax.dot_general(
        gw, x, (((1,), (1,)), ((), ())), preferred_element_type=jnp.float32)
    m = jnp.max(lt, axis=0, keepdims=True)
    ex = jnp.exp(lt - m)
    p = ex / jnp.sum(ex, axis=0, keepdims=True)
    iota = lax.broadcasted_iota(jnp.int32, (NEXP, NTOK), 0).astype(jnp.float32)
    v1 = jnp.max(p, axis=0, keepdims=True)
    e1 = jnp.min(jnp.where(p == v1, iota, float(NEXP)), axis=0, keepdims=True)
    oh1 = (iota == e1).astype(jnp.float32)
    p2 = jnp.where(oh1 > 0, -1.0, p)
    v2 = jnp.max(p2, axis=0, keepdims=True)
    e2 = jnp.min(jnp.where(p2 == v2, iota, float(NEXP)), axis=0, keepdims=True)
    oh2 = (iota == e2).astype(jnp.float32)
    sw = v1 + v2
    w_ref[0:1, :] = v1 / sw
    w_ref[1:2, :] = v2 / sw

    ch = 128
    su = (lax.broadcasted_iota(jnp.int32, (ch, ch), 0)
          < lax.broadcasted_iota(jnp.int32, (ch, ch), 1)).astype(jnp.float32)

    def excl_cumsum(oh):
        parts = []
        carry = jnp.zeros((NEXP, 1), jnp.float32)
        for j in range(NTOK // ch):
            c = oh[:, j * ch:(j + 1) * ch]
            pref = lax.dot_general(c, su, (((1,), (0,)), ((), ())),
                                   preferred_element_type=jnp.float32)
            parts.append(pref + carry)
            carry = carry + jnp.sum(c, axis=1, keepdims=True)
        return jnp.concatenate(parts, axis=1), carry

    r1, cnt1 = excl_cumsum(oh1)
    r2, cnt2 = excl_cumsum(oh2)
    r2 = r2 + cnt1
    counts = cnt1 + cnt2
    pb = jnp.floor((counts + float(BLK - 1)) * (1.0 / BLK))
    ls = (lax.broadcasted_iota(jnp.int32, (NEXP, NEXP), 1)
          < lax.broadcasted_iota(jnp.int32, (NEXP, NEXP), 0)).astype(jnp.float32)
    sb = lax.dot_general(ls, pb, (((1,), (0,)), ((), ())),
                         preferred_element_type=jnp.float32)
    start_slot = sb * float(BLK)
    pos_a = jnp.sum(oh1 * (start_slot + r1), axis=0, keepdims=True)
    pos_b = jnp.sum(oh2 * (start_slot + r2), axis=0, keepdims=True)
    pos_ref[0:1, :] = pos_a.astype(jnp.int32)
    pos_ref[1:2, :] = pos_b.astype(jnp.int32)

    used = jnp.sum(pb, axis=0, keepdims=True)
    bio = lax.broadcasted_iota(jnp.int32, (1, NBLK), 1).astype(jnp.float32)
    bcl = jnp.minimum(bio, used - 1.0)
    eid = jnp.sum((sb <= bcl).astype(jnp.float32), axis=0, keepdims=True) - 1.0
    meta_ref[0:1, 0:NBLK] = eid.astype(jnp.int32)
    meta_ref[0:1, NBLK:NBLK + 1] = used.astype(jnp.int32)


def _router(x, gate_w):
    return pl.pallas_call(
        _router_body,
        out_shape=[
            jax.ShapeDtypeStruct((NTOK, NEXP), jnp.float32),
            jax.ShapeDtypeStruct((TOPK, NTOK), jnp.float32),
            jax.ShapeDtypeStruct((TOPK, NTOK), jnp.int32),
            jax.ShapeDtypeStruct((1, NBLK + 1), jnp.int32),
        ],
    )(x, gate_w)


def _ffn_body(meta_ref, xs_ref, ws_ref, wg_ref, wu_ref, wd_ref, ys_ref):
    b = pl.program_id(0)
    used = meta_ref[NBLK]

    @pl.when(b < used)
    def _():
        xb = xs_ref[...].astype(jnp.bfloat16)
        g = lax.dot_general(xb, wg_ref[0].astype(jnp.bfloat16),
                            (((1,), (1,)), ((), ())),
                            preferred_element_type=jnp.float32)
        u = lax.dot_general(xb, wu_ref[0].astype(jnp.bfloat16),
                            (((1,), (1,)), ((), ())),
                            preferred_element_type=jnp.float32)
        h = (g * lax.logistic(g) * u).astype(jnp.bfloat16)
        part = lax.dot_general(h, wd_ref[0].astype(jnp.bfloat16),
                               (((1,), (1,)), ((), ())),
                               preferred_element_type=jnp.float32)
        ys_ref[...] = part * ws_ref[:, 0:1]


def _ffn(meta, xs, ws2d, wg, wu, wd):
    def bmap(b, m):
        return (jnp.minimum(b, m[NBLK] - 1), 0)

    def wmap_g(b, m):
        return (m[b], 0, 0)

    def wmap_d(b, m):
        return (m[b], 0, 0)

    grid_spec = pltpu.PrefetchScalarGridSpec(
        num_scalar_prefetch=1,
        grid=(NBLK,),
        in_specs=[
            pl.BlockSpec((BLK, HID), bmap),
            pl.BlockSpec((BLK, 128), bmap),
            pl.BlockSpec((1, INTER, HID), wmap_g),
            pl.BlockSpec((1, INTER, HID), wmap_g),
            pl.BlockSpec((1, HID, INTER), wmap_d),
        ],
        out_specs=pl.BlockSpec((BLK, HID), bmap),
    )
    return pl.pallas_call(
        _ffn_body,
        grid_spec=grid_spec,
        out_shape=jax.ShapeDtypeStruct((PTOT, HID), jnp.float32),
    )(meta, xs, ws2d, wg, wu, wd)


def _dispatch_body(x_hbm, w16_hbm, pos_hbm, xs_hbm, ws_hbm,
                   idx_v, rows_v, wrow_v, sem1, sem2):
    wid = lax.axis_index("s") * 2 + lax.axis_index("c")
    base = wid * (NPAIR // NWORK)
    for j in range(NPAIR // NWORK // CHD):
        pbase = base + j * CHD
        pltpu.sync_copy(pos_hbm.at[pl.ds(pbase, CHD)], idx_v)
        tb = lax.rem(pbase, NTOK)
        pltpu.sync_copy(x_hbm.at[pl.ds(tb, CHD)], rows_v)
        pltpu.sync_copy(w16_hbm.at[pl.ds(pbase, CHD)], wrow_v)
        c1 = pltpu.async_copy(rows_v, xs_hbm.at[idx_v], sem1)
        c2 = pltpu.async_copy(wrow_v, ws_hbm.at[idx_v], sem2)
        c1.wait()
        c2.wait()


def _dispatch(x, w16, pos):
    mesh = plsc.VectorSubcoreMesh(core_axis_name="c", subcore_axis_name="s")
    call = pl.kernel(
        _dispatch_body,
        mesh=mesh,
        out_type=[
            jax.ShapeDtypeStruct((PTOT, HID), jnp.float32),
            jax.ShapeDtypeStruct((PTOT, 128), jnp.float32),
        ],
        scratch_types=[
            pltpu.VMEM((CHD,), jnp.int32),
            pltpu.VMEM((CHD, HID), jnp.float32),
            pltpu.VMEM((CHD, 128), jnp.float32),
            pltpu.SemaphoreType.DMA,
            pltpu.SemaphoreType.DMA,
        ],
    )
    return call(x, w16, pos)


def _combine_body(ys_hbm, pos_hbm, out_hbm, idx0_v, idx1_v, t0_v, t1_v, sem):
    wid = lax.axis_index("s") * 2 + lax.axis_index("c")
    base = wid * (NTOK // NWORK)
    for j in range(NTOK // NWORK // CHC):
        tb = base + j * CHC
        pltpu.sync_copy(pos_hbm.at[pl.ds(tb, CHC)], idx0_v)
        pltpu.sync_copy(pos_hbm.at[pl.ds(NTOK + tb, CHC)], idx1_v)
        pltpu.async_copy(ys_hbm.at[idx0_v], t0_v, sem).wait()
        pltpu.async_copy(ys_hbm.at[idx1_v], t1_v, sem).wait()

        @pl.loop(0, CHC)
        def _(r):
            @pl.loop(0, HID, step=16)
            def _(cc):
                slc = (pl.ds(r, 1), pl.ds(cc, 16))
                t0_v.at[*slc][...] = t0_v.at[*slc][...] + t1_v.at[*slc][...]

        pltpu.sync_copy(t0_v, out_hbm.at[pl.ds(tb, CHC)])


def _combine(ys, pos):
    mesh = plsc.VectorSubcoreMesh(core_axis_name="c", subcore_axis_name="s")
    call = pl.kernel(
        _combine_body,
        mesh=mesh,
        out_type=jax.ShapeDtypeStruct((NTOK, HID), jnp.float32),
        scratch_types=[
            pltpu.VMEM((CHC,), jnp.int32),
            pltpu.VMEM((CHC,), jnp.int32),
            pltpu.VMEM((CHC, HID), jnp.float32),
            pltpu.VMEM((CHC, HID), jnp.float32),
            pltpu.SemaphoreType.DMA,
        ],
    )
    return call(ys, pos)


def kernel(hidden_states, gate_w, wg, wu, wd):
    b, s, h = hidden_states.shape
    x = hidden_states.reshape(s, h)
    logits, wrow, posrow, meta = _router(x, gate_w)
    pos = posrow.reshape(NPAIR)
    w16 = jnp.broadcast_to(wrow.reshape(NPAIR, 1), (NPAIR, 128))
    xs, ws2d = _dispatch(x, w16, pos)
    ys = _ffn(meta.reshape(NBLK + 1), xs, ws2d, wg, wu, wd)
    final = _combine(ys, pos)
    return final.reshape(b, s, h), logits
